# Initial kernel scaffold; baseline (speedup 1.0000x reference)
import jax
import jax.numpy as jnp
from jax.experimental import pallas as pl


def kernel(input_tokens, table):
    raise NotImplementedError("write your pallas kernel here")


# speedup vs baseline: 1.1030x; 1.1030x over previous
import functools

import jax
import jax.numpy as jnp
from jax import lax
from jax.experimental import pallas as pl
from jax.experimental.pallas import tpu as pltpu
from jax.experimental.pallas import tpu_sc as plsc

_EMBED_DIM = 32
_NC = 2
_NS = 16
_NW = _NC * _NS


def _sc_gather(table, idx_flat, chunk):
  n = idx_flat.shape[0]
  assert n % (_NW * chunk) == 0
  b_per_w = n // _NW
  n_chunks = b_per_w // chunk
  mesh = plsc.VectorSubcoreMesh(core_axis_name="c", subcore_axis_name="s")

  @functools.partial(
      pl.kernel,
      mesh=mesh,
      out_type=jax.ShapeDtypeStruct((n, _EMBED_DIM), jnp.float32),
      compiler_params=pltpu.CompilerParams(use_tc_tiling_on_sc=False),
      scratch_types=[
          pltpu.VMEM((chunk,), jnp.int32),
          pltpu.VMEM((chunk, _EMBED_DIM), jnp.float32),
          pltpu.SemaphoreType.DMA,
      ],
  )
  def k(table_hbm, idx_hbm, out_hbm, idx_v, rows_v, sem):
    wid = lax.axis_index("s") * _NC + lax.axis_index("c")
    base = wid * b_per_w

    def body(i, _):
      off = base + i * chunk
      pltpu.sync_copy(idx_hbm.at[pl.ds(off, chunk)], idx_v)
      pltpu.async_copy(table_hbm.at[idx_v], rows_v, sem).wait()
      pltpu.sync_copy(rows_v, out_hbm.at[pl.ds(off, chunk)])
      return ()

    lax.fori_loop(0, n_chunks, body, ())

  return k(table, idx_flat)


def kernel(input_tokens, table):
  batch, hist = input_tokens.shape
  idx_flat = input_tokens.reshape(-1).astype(jnp.int32)
  out = _sc_gather(table, idx_flat, chunk=1600)
  return out.reshape(batch, hist, _EMBED_DIM)


# speedup vs baseline: 1.1131x; 1.0091x over previous
import functools

import jax
import jax.numpy as jnp
from jax import lax
from jax.experimental import pallas as pl
from jax.experimental.pallas import tpu as pltpu
from jax.experimental.pallas import tpu_sc as plsc

_EMBED_DIM = 32
_NC = 2
_NS = 16
_NW = _NC * _NS
_NBUF = 4
_LOOKAHEAD = 2


def _sc_gather(table, idx_flat, chunk):
  n = idx_flat.shape[0]
  assert n % (_NW * chunk * _NBUF) == 0
  b_per_w = n // _NW
  n_chunks = b_per_w // chunk
  n_outer = n_chunks // _NBUF
  mesh = plsc.VectorSubcoreMesh(core_axis_name="c", subcore_axis_name="s")

  @functools.partial(
      pl.kernel,
      mesh=mesh,
      out_type=jax.ShapeDtypeStruct((n, _EMBED_DIM), jnp.float32),
      compiler_params=pltpu.CompilerParams(use_tc_tiling_on_sc=False),
      scratch_types=[
          pltpu.VMEM((b_per_w,), jnp.int32),
          *[pltpu.VMEM((chunk, _EMBED_DIM), jnp.float32) for _ in range(_NBUF)],
          *[pltpu.SemaphoreType.DMA for _ in range(2 * _NBUF)],
      ],
  )
  def k(table_hbm, idx_hbm, out_hbm, idx_v, *bufs_and_sems):
    rows = bufs_and_sems[:_NBUF]
    sg = bufs_and_sems[_NBUF:2 * _NBUF]
    sw = bufs_and_sems[2 * _NBUF:]
    wid = lax.axis_index("s") * _NC + lax.axis_index("c")
    base = wid * b_per_w

    pltpu.sync_copy(idx_hbm.at[pl.ds(base, b_per_w)], idx_v)

    def gather_desc(g, b):
      return pltpu.make_async_copy(
          table_hbm.at[idx_v.at[pl.ds(g * chunk, chunk)]], rows[b], sg[b])

    def wb_desc(g, b):
      return pltpu.make_async_copy(
          rows[b], out_hbm.at[pl.ds(base + g * chunk, chunk)], sw[b])

    for b in range(_LOOKAHEAD):
      gather_desc(b, b).start()

    def body(t, _):
      for b in range(_NBUF):
        g = t * _NBUF + b
        nxt = g + _LOOKAHEAD
        nb = (b + _LOOKAHEAD) % _NBUF
        prev = nxt - _NBUF

        @pl.when(jnp.logical_and(nxt < n_chunks, prev >= 0))
        def _():
          wb_desc(prev, nb).wait()

        @pl.when(nxt < n_chunks)
        def _():
          gather_desc(nxt, nb).start()

        gather_desc(g, b).wait()
        wb_desc(g, b).start()
      return ()

    lax.fori_loop(0, n_outer, body, ())

    for g in range(n_chunks - (_NBUF - _LOOKAHEAD), n_chunks):
      wb_desc(g, g % _NBUF).wait()

  return k(table, idx_flat)


def kernel(input_tokens, table):
  batch, hist = input_tokens.shape
  idx_flat = input_tokens.reshape(-1).astype(jnp.int32)
  out = _sc_gather(table, idx_flat, chunk=800)
  return out.reshape(batch, hist, _EMBED_DIM)


# speedup vs baseline: 1.8109x; 1.6269x over previous
import functools

import jax
import jax.numpy as jnp
from jax import lax
from jax.experimental import pallas as pl
from jax.experimental.pallas import tpu as pltpu
from jax.experimental.pallas import tpu_sc as plsc

_EMBED_DIM = 32
_NC = 2
_NS = 16
_NW = _NC * _NS
_NBUF = 4
_LOOKAHEAD = 2
_CHUNK = 800


def _sc_gather(table, idx32, batch, hist):
  n = batch * hist
  b_per_w = n // _NW
  rows_per_w = batch // _NW
  n_chunks = b_per_w // _CHUNK
  n_outer = n_chunks // _NBUF
  rows_per_chunk = _CHUNK // hist
  assert n_chunks % _NBUF == 0 and _CHUNK % hist == 0
  mesh = plsc.VectorSubcoreMesh(core_axis_name="c", subcore_axis_name="s")

  @functools.partial(
      pl.kernel,
      mesh=mesh,
      out_type=jax.ShapeDtypeStruct((batch, hist, _EMBED_DIM), jnp.float32),
      compiler_params=pltpu.CompilerParams(use_tc_tiling_on_sc=False),
      scratch_types=[
          pltpu.VMEM((b_per_w,), jnp.int32),
          *[pltpu.VMEM((_CHUNK, _EMBED_DIM), jnp.float32) for _ in range(_NBUF)],
          *[pltpu.SemaphoreType.DMA for _ in range(2 * _NBUF)],
      ],
  )
  def k(table_hbm, idx_hbm, out_hbm, idx_v, *bufs_and_sems):
    rows = bufs_and_sems[:_NBUF]
    sg = bufs_and_sems[_NBUF:2 * _NBUF]
    sw = bufs_and_sems[2 * _NBUF:]
    wid = lax.axis_index("s") * _NC + lax.axis_index("c")
    row_base = wid * rows_per_w

    pltpu.sync_copy(idx_hbm.at[wid], idx_v)

    def gather_desc(g, b):
      return pltpu.make_async_copy(
          table_hbm.at[idx_v.at[pl.ds(g * _CHUNK, _CHUNK)]], rows[b], sg[b])

    def wb_descs(g, b):
      for j in range(rows_per_chunk):
        yield pltpu.make_async_copy(
            rows[b].at[pl.ds(j * hist, hist)],
            out_hbm.at[row_base + g * rows_per_chunk + j],
            sw[b])

    for b in range(_LOOKAHEAD):
      gather_desc(b, b).start()

    def body(t, _):
      for b in range(_NBUF):
        g = t * _NBUF + b
        nxt = g + _LOOKAHEAD
        nb = (b + _LOOKAHEAD) % _NBUF
        prev = nxt - _NBUF

        @pl.when(jnp.logical_and(nxt < n_chunks, prev >= 0))
        def _():
          for d in wb_descs(prev, nb):
            d.wait()

        @pl.when(nxt < n_chunks)
        def _():
          gather_desc(nxt, nb).start()

        gather_desc(g, b).wait()
        for d in wb_descs(g, b):
          d.start()
      return ()

    lax.fori_loop(0, n_outer, body, ())

    for g in range(n_chunks - (_NBUF - _LOOKAHEAD), n_chunks):
      for d in wb_descs(g, g % _NBUF):
        d.wait()

  return k(table, idx32)


def kernel(input_tokens, table):
  batch, hist = input_tokens.shape
  idx32 = input_tokens.reshape(_NW, (batch * hist) // _NW).astype(jnp.int32)
  return _sc_gather(table, idx32, batch, hist)
